# baseline (speedup 1.0000x reference)
import functools

import jax
import jax.numpy as jnp
from jax import lax
from jax.experimental import pallas as pl
from jax.experimental.pallas import tpu as pltpu
from jax.experimental.pallas import tpu_sc as plsc

EC = 128
SE = 16
NS = 16


def _rsqrt_newton(d):
    bits = plsc.bitcast(d, jnp.int32)
    bits = jnp.int32(0x5F3759DF) - (bits >> 1)
    y = plsc.bitcast(bits, jnp.float32)
    for _ in range(3):
        y = y * (1.5 - 0.5 * d * y * y)
    return jnp.where(d > 0.5, y, 0.0)


def _iota16():
    return lax.broadcasted_iota(jnp.int32, (16,), 0)


def _make_prep_kernel(N_pad, HC, E_pad, RT):
    mesh = plsc.VectorSubcoreMesh(core_axis_name="c", subcore_axis_name="s")
    rows_t = E_pad // EC // NS
    RB = 128
    n_rchunk = RT // RB
    ndr8 = (RT // 128 + 7) // 8 * 8

    @functools.partial(
        pl.kernel,
        mesh=mesh,
        compiler_params=pltpu.CompilerParams(
            needs_layout_passes=False, use_tc_tiling_on_sc=False),
        out_type=(
            jax.ShapeDtypeStruct((NS * ndr8, 128), jnp.float32),
            jax.ShapeDtypeStruct((N_pad, HC), jnp.float32),
            jax.ShapeDtypeStruct((N_pad, HC), jnp.float32),
        ),
        scratch_types=[
            pltpu.VMEM((N_pad,), jnp.float32),
            pltpu.VMEM_SHARED((NS, N_pad), jnp.float32),
            pltpu.VMEM((EC,), jnp.int32),
            pltpu.VMEM((NS, RT), jnp.float32),
            pltpu.VMEM((ndr8, 128), jnp.float32),
            pltpu.VMEM((RB, HC), jnp.float32),
            pltpu.VMEM((RB, HC), jnp.float32),
        ],
    )
    def prep(featL_hbm, featR_hbm, dst2d_hbm, dinv_hbm, y0L_hbm, y0R_hbm,
             deg_v, shr, dbuf, cbuf, dch, rbuf, rbuf2):
        cid = lax.axis_index("c")
        sid = lax.axis_index("s")

        @pl.loop(0, N_pad // 16)
        def _(i):
            deg_v[pl.ds(i * 16, 16)] = jnp.zeros((16,), jnp.float32)

        row0 = sid * rows_t
        ones = jnp.ones((16,), jnp.float32)

        @pl.loop(0, rows_t)
        def _(j):
            pltpu.sync_copy(dst2d_hbm.at[row0 + j], dbuf)
            for g in range(EC // 16):
                idx = dbuf[pl.ds(g * 16, 16)]
                plsc.addupdate_scatter(deg_v, [idx], ones)

        pltpu.sync_copy(deg_v, shr.at[sid])
        plsc.subcore_barrier()

        nb = sid * RT
        pltpu.sync_copy(shr.at[:, pl.ds(nb, RT)], cbuf)

        @pl.loop(0, ndr8)
        def _(r):
            for j in range(128 // 16):
                dch[r, pl.ds(j * 16, 16)] = jnp.zeros((16,), jnp.float32)

        @pl.loop(0, RT // 16)
        def _(j):
            acc = jnp.zeros((16,), jnp.float32)
            for r in range(NS):
                acc = acc + cbuf[r, pl.ds(j * 16, 16)]
            flat = j * 16 + _iota16()
            plsc.store_scatter(dch, [flat >> 7, flat & 127],
                               _rsqrt_newton(acc))

        @pl.when(cid == 0)
        def _():
            pltpu.sync_copy(dch, dinv_hbm.at[pl.ds(sid * ndr8, ndr8)])

        def scale_half(feat_ref, yout_ref):
            @pl.loop(0, n_rchunk)
            def _(c):
                r0 = nb + c * RB
                pltpu.sync_copy(feat_ref.at[pl.ds(r0, RB)], rbuf)

                @pl.loop(0, RB // 16)
                def _(g):
                    flat = c * RB + g * 16 + _iota16()
                    dvec = plsc.load_gather(dch, [flat >> 7, flat & 127])
                    for r16 in range(16):
                        s = dvec[r16]
                        row = g * 16 + r16
                        for j in range(HC // 16):
                            rbuf2[row, pl.ds(j * 16, 16)] = (
                                rbuf[row, pl.ds(j * 16, 16)] * s)

                pltpu.sync_copy(rbuf2, yout_ref.at[pl.ds(r0, RB)])

        @pl.when(cid == 0)
        def _():
            scale_half(featL_hbm, y0L_hbm)

        @pl.when(cid == 1)
        def _():
            scale_half(featR_hbm, y0R_hbm)

    return prep


def _make_hop_kernel(N_pad, HC, E_pad, RT):
    mesh = plsc.VectorSubcoreMesh(core_axis_name="c", subcore_axis_name="s")
    rows_t = E_pad // EC // NS
    RB = 128
    n_rchunk = RT // RB
    ndr8 = (RT // 128 + 7) // 8 * 8

    @functools.partial(
        pl.kernel,
        mesh=mesh,
        compiler_params=pltpu.CompilerParams(
            needs_layout_passes=False, use_tc_tiling_on_sc=False),
        out_type=(
            jax.ShapeDtypeStruct((N_pad, HC), jnp.float32),
            jax.ShapeDtypeStruct((N_pad, HC), jnp.float32),
            jax.ShapeDtypeStruct((N_pad, HC), jnp.float32),
            jax.ShapeDtypeStruct((N_pad, HC), jnp.float32),
        ),
        scratch_types=[
            pltpu.VMEM_SHARED((N_pad, HC), jnp.float32),
            pltpu.VMEM((1, EC, HC), jnp.float32),
            pltpu.VMEM((1, EC, HC), jnp.float32),
            pltpu.VMEM((1, EC, HC), jnp.float32),
            pltpu.VMEM((1, EC, HC), jnp.float32),
            pltpu.VMEM((2, SE, EC), jnp.int32),
            pltpu.VMEM((2, SE, EC), jnp.int32),
            pltpu.VMEM((ndr8, 128), jnp.float32),
            pltpu.SemaphoreType.DMA,
            pltpu.SemaphoreType.DMA,
            pltpu.SemaphoreType.DMA,
            pltpu.SemaphoreType.DMA,
            pltpu.SemaphoreType.DMA,
            pltpu.SemaphoreType.DMA,
            pltpu.SemaphoreType.DMA,
            pltpu.SemaphoreType.DMA,
        ],
    )
    def hop(yL_hbm, yR_hbm, src2d_hbm, dst2d_hbm, dinv_hbm,
            xL_out, xR_out, yLn_out, yRn_out,
            acc, b0, b1, b2, b3, srcI, dstI, dch,
            sg0, sg1, sg2, sg3, ss0, ss1, ss2, ss3):
        cid = lax.axis_index("c")
        sid = lax.axis_index("s")
        bufs = [b0, b1, b2, b3]
        semG = [sg0, sg1, sg2, sg3]
        semS = [ss0, ss1, ss2, ss3]

        z0 = b0.at[0]

        @pl.loop(0, RB)
        def _(r):
            for j in range(HC // 16):
                z0[r, pl.ds(j * 16, 16)] = jnp.zeros((16,), jnp.float32)

        @pl.loop(0, RT // RB)
        def _(c):
            pltpu.sync_copy(z0, acc.at[pl.ds(sid * RT + c * RB, RB)])

        plsc.subcore_barrier()

        br = sid * rows_t

        def load_super(s):
            slot = lax.rem(s, 2)
            pltpu.sync_copy(src2d_hbm.at[pl.ds(br + s * SE, SE)],
                            srcI.at[slot])
            pltpu.sync_copy(dst2d_hbm.at[pl.ds(br + s * SE, SE)],
                            dstI.at[slot])

        def sidx(c):
            return lax.rem(lax.div(c, SE), 2), lax.rem(c, SE)

        n_chunk = rows_t

        def edge_phase(y_ref):
            def startG(p, c):
                sl, ro = sidx(c)
                pltpu.async_copy(y_ref.at[srcI.at[sl, ro]], bufs[p].at[0],
                                 semG[p])

            def waitG(p, c):
                sl, ro = sidx(c)
                pltpu.make_async_copy(
                    y_ref.at[srcI.at[sl, ro]], bufs[p].at[0], semG[p]).wait()

            def startS(p, c):
                sl, ro = sidx(c)
                pltpu.async_copy(bufs[p].at[0], acc.at[dstI.at[sl, ro]],
                                 semS[p], add=True)

            def waitS(p, c):
                sl, ro = sidx(c)
                pltpu.make_async_copy(
                    bufs[p].at[0], acc.at[dstI.at[sl, ro]], semS[p]).wait()

            load_super(0)
            startG(0, 0)
            startG(1, 1)
            waitG(0, 0)
            startS(0, 0)
            startG(2, 2)
            waitG(1, 1)
            startS(1, 1)
            startG(3, 3)

            @pl.loop(0, (n_chunk - 4) // 4)
            def _(q):
                for t in range(4):
                    c = 4 * q + 2 + t
                    p = (2 + t) % 4
                    waitG(p, c)
                    startS(p, c)
                    waitS((p + 2) % 4, c - 2)

                    @pl.when(lax.rem(c + 2, SE) == 0)
                    def _():
                        load_super(lax.div(c + 2, SE))

                    startG((p + 2) % 4, c + 2)

            last = n_chunk - 2
            waitG(2, last)
            startS(2, last)
            waitS(0, last - 2)
            waitG(3, last + 1)
            startS(3, last + 1)
            waitS(1, last - 1)
            waitS(2, last)
            waitS(3, last + 1)

        @pl.when(cid == 0)
        def _():
            edge_phase(yL_hbm)

        @pl.when(cid == 1)
        def _():
            edge_phase(yR_hbm)

        plsc.subcore_barrier()

        nb = sid * RT
        pltpu.sync_copy(dinv_hbm.at[pl.ds(sid * ndr8, ndr8)], dch)

        def rescale(x_ref, y_ref):
            b0v = b0.at[0]
            b1v = b1.at[0]

            @pl.loop(0, n_rchunk)
            def _(c):
                r0 = nb + c * RB
                pltpu.sync_copy(acc.at[pl.ds(r0, RB)], b0v)

                @pl.loop(0, RB // 16)
                def _(g):
                    flat = c * RB + g * 16 + _iota16()
                    dvec = plsc.load_gather(dch, [flat >> 7, flat & 127])
                    for r16 in range(16):
                        s = dvec[r16]
                        row = g * 16 + r16
                        for j in range(HC // 16):
                            b1v[row, pl.ds(j * 16, 16)] = (
                                b0v[row, pl.ds(j * 16, 16)] * s)

                pltpu.sync_copy(b1v, x_ref.at[pl.ds(r0, RB)])

                @pl.loop(0, RB // 16)
                def _(g):
                    flat = c * RB + g * 16 + _iota16()
                    dvec = plsc.load_gather(dch, [flat >> 7, flat & 127])
                    for r16 in range(16):
                        s = dvec[r16]
                        row = g * 16 + r16
                        for j in range(HC // 16):
                            b0v[row, pl.ds(j * 16, 16)] = (
                                b1v[row, pl.ds(j * 16, 16)] * s)

                pltpu.sync_copy(b0v, y_ref.at[pl.ds(r0, RB)])

        @pl.when(cid == 0)
        def _():
            rescale(xL_out, yLn_out)

        @pl.when(cid == 1)
        def _():
            rescale(xR_out, yRn_out)

    return hop


def _mlp_body(*refs):
    xs_refs = refs[:-7]
    w1_ref, b1_ref, gs_ref, beta_ref, w2_ref, b2_ref, o_ref = refs[-7:]
    dn = (((1,), (1,)), ((), ()))
    h = None
    for i, x_ref in enumerate(xs_refs):
        k, c = divmod(i, 2)
        t = lax.dot_general(x_ref[...], w1_ref[k, c], dn,
                            preferred_element_type=jnp.float32)
        h = t if h is None else h + t
    h = h + b1_ref[...]
    h = jnp.where(h > 0, h, 0.2 * h)
    h = h * gs_ref[...] + beta_ref[...]
    o = lax.dot_general(h, w2_ref[...], dn,
                        preferred_element_type=jnp.float32) + b2_ref[...]
    o_ref[...] = o


def kernel(feature, edge_index, W1, b1, gamma, beta, W2, b2):
    N, D = feature.shape
    E = edge_index.shape[1]
    H, fan1 = W1.shape
    K = fan1 // D - 1
    O = W2.shape[0]
    HC = D // 2

    RT = ((N + NS - 1) // NS + 127) // 128 * 128
    N_pad = RT * NS

    unit = NS * SE * EC
    E_pad = (E + unit - 1) // unit * unit
    pad_e = E_pad - E
    src_p = jnp.concatenate([edge_index[0], jnp.zeros((pad_e,), jnp.int32)])
    dst_p = jnp.concatenate(
        [edge_index[1], jnp.full((pad_e,), N_pad - 1, jnp.int32)])

    feat_pad = jnp.pad(feature, ((0, N_pad - N), (0, 0)))
    featL = feat_pad[:, :HC]
    featR = feat_pad[:, HC:]
    src2d = src_p.reshape(E_pad // EC, EC)
    dst2d = dst_p.reshape(E_pad // EC, EC)

    prep = _make_prep_kernel(N_pad, HC, E_pad, RT)
    hop = _make_hop_kernel(N_pad, HC, E_pad, RT)

    dinv, yL, yR = prep(featL, featR, dst2d)
    xs = [featL, featR]
    for _ in range(K):
        xL, xR, yL, yR = hop(yL, yR, src2d, dst2d, dinv)
        xs.extend([xL, xR])

    W1r = W1.reshape(H, K + 1, 2, HC).transpose(1, 2, 0, 3)
    gs = (gamma / jnp.sqrt(1.0 + 1e-5)).reshape(1, H)
    b1r = b1.reshape(1, H)
    betar = beta.reshape(1, H)
    b2r = b2.reshape(1, O)

    BM = 256
    grid = (N_pad // BM,)
    x_spec = pl.BlockSpec((BM, HC), lambda i: (i, 0))
    out = pl.pallas_call(
        _mlp_body,
        grid=grid,
        in_specs=[x_spec] * (2 * (K + 1)) + [
            pl.BlockSpec((K + 1, 2, H, HC), lambda i: (0, 0, 0, 0)),
            pl.BlockSpec((1, H), lambda i: (0, 0)),
            pl.BlockSpec((1, H), lambda i: (0, 0)),
            pl.BlockSpec((1, H), lambda i: (0, 0)),
            pl.BlockSpec((O, H), lambda i: (0, 0)),
            pl.BlockSpec((1, O), lambda i: (0, 0)),
        ],
        out_specs=pl.BlockSpec((BM, O), lambda i: (i, 0)),
        out_shape=jax.ShapeDtypeStruct((N_pad, O), jnp.float32),
    )(*xs, W1r, b1r, gs, betar, W2, b2r)
    return out[:N]


# speedup vs baseline: 1.0348x; 1.0348x over previous
import functools

import jax
import jax.numpy as jnp
from jax import lax
from jax.experimental import pallas as pl
from jax.experimental.pallas import tpu as pltpu
from jax.experimental.pallas import tpu_sc as plsc

EC = 256
SE = 8
NS = 16


def _rsqrt_newton(d):
    bits = plsc.bitcast(d, jnp.int32)
    bits = jnp.int32(0x5F3759DF) - (bits >> 1)
    y = plsc.bitcast(bits, jnp.float32)
    for _ in range(3):
        y = y * (1.5 - 0.5 * d * y * y)
    return jnp.where(d > 0.5, y, 0.0)


def _iota16():
    return lax.broadcasted_iota(jnp.int32, (16,), 0)


def _make_prep_kernel(N_pad, HC, E_pad, RT):
    mesh = plsc.VectorSubcoreMesh(core_axis_name="c", subcore_axis_name="s")
    rows_t = E_pad // EC // NS
    RB = 128
    n_rchunk = RT // RB
    ndr8 = (RT // 128 + 7) // 8 * 8

    @functools.partial(
        pl.kernel,
        mesh=mesh,
        compiler_params=pltpu.CompilerParams(
            needs_layout_passes=False, use_tc_tiling_on_sc=False),
        out_type=(
            jax.ShapeDtypeStruct((NS * ndr8, 128), jnp.float32),
            jax.ShapeDtypeStruct((N_pad, HC), jnp.float32),
            jax.ShapeDtypeStruct((N_pad, HC), jnp.float32),
        ),
        scratch_types=[
            pltpu.VMEM((N_pad,), jnp.float32),
            pltpu.VMEM_SHARED((NS, N_pad), jnp.float32),
            pltpu.VMEM((EC,), jnp.int32),
            pltpu.VMEM((NS, RT), jnp.float32),
            pltpu.VMEM((ndr8, 128), jnp.float32),
            pltpu.VMEM((RB, HC), jnp.float32),
            pltpu.VMEM((RB, HC), jnp.float32),
        ],
    )
    def prep(featL_hbm, featR_hbm, dst2d_hbm, dinv_hbm, y0L_hbm, y0R_hbm,
             deg_v, shr, dbuf, cbuf, dch, rbuf, rbuf2):
        cid = lax.axis_index("c")
        sid = lax.axis_index("s")

        @pl.loop(0, N_pad // 16)
        def _(i):
            deg_v[pl.ds(i * 16, 16)] = jnp.zeros((16,), jnp.float32)

        row0 = sid * rows_t
        ones = jnp.ones((16,), jnp.float32)

        @pl.loop(0, rows_t)
        def _(j):
            pltpu.sync_copy(dst2d_hbm.at[row0 + j], dbuf)
            for g in range(EC // 16):
                idx = dbuf[pl.ds(g * 16, 16)]
                plsc.addupdate_scatter(deg_v, [idx], ones)

        pltpu.sync_copy(deg_v, shr.at[sid])
        plsc.subcore_barrier()

        nb = sid * RT
        pltpu.sync_copy(shr.at[:, pl.ds(nb, RT)], cbuf)

        @pl.loop(0, ndr8)
        def _(r):
            for j in range(128 // 16):
                dch[r, pl.ds(j * 16, 16)] = jnp.zeros((16,), jnp.float32)

        @pl.loop(0, RT // 16)
        def _(j):
            acc = jnp.zeros((16,), jnp.float32)
            for r in range(NS):
                acc = acc + cbuf[r, pl.ds(j * 16, 16)]
            flat = j * 16 + _iota16()
            plsc.store_scatter(dch, [flat >> 7, flat & 127],
                               _rsqrt_newton(acc))

        @pl.when(cid == 0)
        def _():
            pltpu.sync_copy(dch, dinv_hbm.at[pl.ds(sid * ndr8, ndr8)])

        def scale_half(feat_ref, yout_ref):
            @pl.loop(0, n_rchunk)
            def _(c):
                r0 = nb + c * RB
                pltpu.sync_copy(feat_ref.at[pl.ds(r0, RB)], rbuf)

                @pl.loop(0, RB // 16)
                def _(g):
                    flat = c * RB + g * 16 + _iota16()
                    dvec = plsc.load_gather(dch, [flat >> 7, flat & 127])
                    for r16 in range(16):
                        s = dvec[r16]
                        row = g * 16 + r16
                        for j in range(HC // 16):
                            rbuf2[row, pl.ds(j * 16, 16)] = (
                                rbuf[row, pl.ds(j * 16, 16)] * s)

                pltpu.sync_copy(rbuf2, yout_ref.at[pl.ds(r0, RB)])

        @pl.when(cid == 0)
        def _():
            scale_half(featL_hbm, y0L_hbm)

        @pl.when(cid == 1)
        def _():
            scale_half(featR_hbm, y0R_hbm)

    return prep


def _make_hop_kernel(N_pad, HC, E_pad, RT):
    mesh = plsc.VectorSubcoreMesh(core_axis_name="c", subcore_axis_name="s")
    rows_t = E_pad // EC // NS
    RB = 128
    n_rchunk = RT // RB
    ndr8 = (RT // 128 + 7) // 8 * 8

    @functools.partial(
        pl.kernel,
        mesh=mesh,
        compiler_params=pltpu.CompilerParams(
            needs_layout_passes=False, use_tc_tiling_on_sc=False),
        out_type=(
            jax.ShapeDtypeStruct((N_pad, HC), jnp.float32),
            jax.ShapeDtypeStruct((N_pad, HC), jnp.float32),
            jax.ShapeDtypeStruct((N_pad, HC), jnp.float32),
            jax.ShapeDtypeStruct((N_pad, HC), jnp.float32),
        ),
        scratch_types=[
            pltpu.VMEM_SHARED((N_pad, HC), jnp.float32),
            pltpu.VMEM((1, EC, HC), jnp.float32),
            pltpu.VMEM((1, EC, HC), jnp.float32),
            pltpu.VMEM((2, SE, EC), jnp.int32),
            pltpu.VMEM((2, SE, EC), jnp.int32),
            pltpu.VMEM((ndr8, 128), jnp.float32),
            pltpu.SemaphoreType.DMA,
            pltpu.SemaphoreType.DMA,
        ],
    )
    def hop(yL_hbm, yR_hbm, src2d_hbm, dst2d_hbm, dinv_hbm,
            xL_out, xR_out, yLn_out, yRn_out,
            acc, b0, b1, srcI, dstI, dch, semA, semB):
        cid = lax.axis_index("c")
        sid = lax.axis_index("s")

        z0 = b0.at[0, pl.ds(0, RB)]

        @pl.loop(0, RB)
        def _(r):
            for j in range(HC // 16):
                z0[r, pl.ds(j * 16, 16)] = jnp.zeros((16,), jnp.float32)

        @pl.loop(0, RT // RB)
        def _(c):
            pltpu.sync_copy(z0, acc.at[pl.ds(sid * RT + c * RB, RB)])

        plsc.subcore_barrier()

        br = sid * rows_t

        def load_super(s):
            slot = lax.rem(s, 2)
            pltpu.sync_copy(src2d_hbm.at[pl.ds(br + s * SE, SE)],
                            srcI.at[slot])
            pltpu.sync_copy(dst2d_hbm.at[pl.ds(br + s * SE, SE)],
                            dstI.at[slot])

        def sidx(c):
            return lax.rem(lax.div(c, SE), 2), lax.rem(c, SE)

        n_chunk = rows_t

        def edge_phase(y_ref):
            def startG(buf, sem, c):
                sl, ro = sidx(c)
                pltpu.async_copy(y_ref.at[srcI.at[sl, ro]], buf.at[0], sem)

            def waitG(buf, sem, c):
                sl, ro = sidx(c)
                pltpu.make_async_copy(
                    y_ref.at[srcI.at[sl, ro]], buf.at[0], sem).wait()

            def scat(buf, c):
                sl, ro = sidx(c)
                pltpu.sync_copy(buf.at[0], acc.at[dstI.at[sl, ro]], add=True)

            load_super(0)
            startG(b0, semA, 0)

            @pl.loop(0, n_chunk // 2)
            def _(j):
                a = 2 * j
                startG(b1, semB, a + 1)
                waitG(b0, semA, a)
                scat(b0, a)

                @pl.when(jnp.logical_and(lax.rem(a + 2, SE) == 0,
                                         a + 2 < n_chunk))
                def _():
                    load_super(lax.div(a + 2, SE))

                @pl.when(a + 2 < n_chunk)
                def _():
                    startG(b0, semA, a + 2)

                waitG(b1, semB, a + 1)
                scat(b1, a + 1)

        @pl.when(cid == 0)
        def _():
            edge_phase(yL_hbm)

        @pl.when(cid == 1)
        def _():
            edge_phase(yR_hbm)

        plsc.subcore_barrier()

        nb = sid * RT
        pltpu.sync_copy(dinv_hbm.at[pl.ds(sid * ndr8, ndr8)], dch)

        def rescale(x_ref, y_ref):
            b0v = b0.at[0, pl.ds(0, RB)]
            b1v = b1.at[0, pl.ds(0, RB)]

            @pl.loop(0, n_rchunk)
            def _(c):
                r0 = nb + c * RB
                pltpu.sync_copy(acc.at[pl.ds(r0, RB)], b0v)

                @pl.loop(0, RB // 16)
                def _(g):
                    flat = c * RB + g * 16 + _iota16()
                    dvec = plsc.load_gather(dch, [flat >> 7, flat & 127])
                    for r16 in range(16):
                        s = dvec[r16]
                        row = g * 16 + r16
                        for j in range(HC // 16):
                            b1v[row, pl.ds(j * 16, 16)] = (
                                b0v[row, pl.ds(j * 16, 16)] * s)

                pltpu.sync_copy(b1v, x_ref.at[pl.ds(r0, RB)])

                @pl.loop(0, RB // 16)
                def _(g):
                    flat = c * RB + g * 16 + _iota16()
                    dvec = plsc.load_gather(dch, [flat >> 7, flat & 127])
                    for r16 in range(16):
                        s = dvec[r16]
                        row = g * 16 + r16
                        for j in range(HC // 16):
                            b0v[row, pl.ds(j * 16, 16)] = (
                                b1v[row, pl.ds(j * 16, 16)] * s)

                pltpu.sync_copy(b0v, y_ref.at[pl.ds(r0, RB)])

        @pl.when(cid == 0)
        def _():
            rescale(xL_out, yLn_out)

        @pl.when(cid == 1)
        def _():
            rescale(xR_out, yRn_out)

    return hop


def _mlp_body(*refs):
    xs_refs = refs[:-7]
    w1_ref, b1_ref, gs_ref, beta_ref, w2_ref, b2_ref, o_ref = refs[-7:]
    dn = (((1,), (1,)), ((), ()))
    h = None
    for i, x_ref in enumerate(xs_refs):
        k, c = divmod(i, 2)
        t = lax.dot_general(x_ref[...], w1_ref[k, c], dn,
                            preferred_element_type=jnp.float32)
        h = t if h is None else h + t
    h = h + b1_ref[...]
    h = jnp.where(h > 0, h, 0.2 * h)
    h = h * gs_ref[...] + beta_ref[...]
    o = lax.dot_general(h, w2_ref[...], dn,
                        preferred_element_type=jnp.float32) + b2_ref[...]
    o_ref[...] = o


def kernel(feature, edge_index, W1, b1, gamma, beta, W2, b2):
    N, D = feature.shape
    E = edge_index.shape[1]
    H, fan1 = W1.shape
    K = fan1 // D - 1
    O = W2.shape[0]
    HC = D // 2

    RT = ((N + NS - 1) // NS + 127) // 128 * 128
    N_pad = RT * NS

    unit = NS * SE * EC
    E_pad = (E + unit - 1) // unit * unit
    pad_e = E_pad - E
    src_p = jnp.concatenate([edge_index[0], jnp.zeros((pad_e,), jnp.int32)])
    dst_p = jnp.concatenate(
        [edge_index[1], jnp.full((pad_e,), N_pad - 1, jnp.int32)])

    feat_pad = jnp.pad(feature, ((0, N_pad - N), (0, 0)))
    featL = feat_pad[:, :HC]
    featR = feat_pad[:, HC:]
    src2d = src_p.reshape(E_pad // EC, EC)
    dst2d = dst_p.reshape(E_pad // EC, EC)

    prep = _make_prep_kernel(N_pad, HC, E_pad, RT)
    hop = _make_hop_kernel(N_pad, HC, E_pad, RT)

    dinv, yL, yR = prep(featL, featR, dst2d)
    xs = [featL, featR]
    for _ in range(K):
        xL, xR, yL, yR = hop(yL, yR, src2d, dst2d, dinv)
        xs.extend([xL, xR])

    W1r = W1.reshape(H, K + 1, 2, HC).transpose(1, 2, 0, 3)
    gs = (gamma / jnp.sqrt(1.0 + 1e-5)).reshape(1, H)
    b1r = b1.reshape(1, H)
    betar = beta.reshape(1, H)
    b2r = b2.reshape(1, O)

    BM = 256
    grid = (N_pad // BM,)
    x_spec = pl.BlockSpec((BM, HC), lambda i: (i, 0))
    out = pl.pallas_call(
        _mlp_body,
        grid=grid,
        in_specs=[x_spec] * (2 * (K + 1)) + [
            pl.BlockSpec((K + 1, 2, H, HC), lambda i: (0, 0, 0, 0)),
            pl.BlockSpec((1, H), lambda i: (0, 0)),
            pl.BlockSpec((1, H), lambda i: (0, 0)),
            pl.BlockSpec((1, H), lambda i: (0, 0)),
            pl.BlockSpec((O, H), lambda i: (0, 0)),
            pl.BlockSpec((1, O), lambda i: (0, 0)),
        ],
        out_specs=pl.BlockSpec((BM, O), lambda i: (i, 0)),
        out_shape=jax.ShapeDtypeStruct((N_pad, O), jnp.float32),
    )(*xs, W1r, b1r, gs, betar, W2, b2r)
    return out[:N]
